# baseline (speedup 1.0000x reference)
import jax
import jax.numpy as jnp
from jax import lax
from jax.experimental import pallas as pl
from jax.experimental.pallas import tpu as pltpu
from jax.experimental.pallas import tpu_sc as plsc

B = 16384
D = 64
E = 1000000
L = 16
NC = 2
NS = 16
NW = NC * NS
PW = B // NW
CH = 128
NCH = PW // CH
G = PW // L


def _rsqrt(x):
    i = lax.bitcast_convert_type(x, jnp.int32)
    i = jnp.int32(0x5F3759DF) - lax.shift_right_arithmetic(i, 1)
    y = lax.bitcast_convert_type(i, jnp.float32)
    for _ in range(3):
        t = (x * y) * y
        y = y * (jnp.float32(1.5) - jnp.float32(0.5) * t)
    return y


def _body(head_hbm, tail_hbm, rel_hbm, ent_hbm, remb_hbm, out_hbm,
          idx_h, idx_t, idx_r, wl_h, wl_t, wl_r,
          h_buf, t_buf, r_buf, out_v, sem):
    wid = lax.axis_index("s") * NC + lax.axis_index("c")

    pltpu.sync_copy(head_hbm.at[wid], idx_h)
    pltpu.sync_copy(tail_hbm.at[wid], idx_t)
    pltpu.sync_copy(rel_hbm.at[wid], idx_r)

    def feat(f, carry):
        off = f * E
        for src, dst in ((idx_h, wl_h), (idx_t, wl_t), (idx_r, wl_r)):
            for c in range(NCH):
                for k in range(CH // L):
                    s = pl.ds(k * L, L)
                    dst[c, s] = src[c, s] + off
        cps = []
        for tab, wl, buf in ((ent_hbm, wl_h, h_buf),
                             (ent_hbm, wl_t, t_buf),
                             (remb_hbm, wl_r, r_buf)):
            for c in range(NCH):
                cps.append(pltpu.async_copy(
                    tab.at[wl.at[c]], buf.at[f, pl.ds(c * CH, CH)], sem))
        for cp in cps:
            cp.wait()
        return carry

    lax.fori_loop(0, D, feat, 0)

    def grp(j, carry):
        sl = pl.ds(j * L, L)

        def facc(f, hh_tt):
            hh, tt = hh_tt
            hv = h_buf[f, sl]
            tv = t_buf[f, sl]
            return hh + hv * hv, tt + tv * tv

        z = jnp.zeros((L,), jnp.float32)
        hh, tt = lax.fori_loop(0, D, facc, (z, z))
        ih = jnp.minimum(_rsqrt(hh), jnp.float32(1e12))
        it = jnp.minimum(_rsqrt(tt), jnp.float32(1e12))

        def fsc(f, acc):
            return acc + jnp.abs(h_buf[f, sl] * ih
                                 + r_buf[f, sl]
                                 - t_buf[f, sl] * it)

        out_v[sl] = lax.fori_loop(0, D, fsc, z)
        return carry

    lax.fori_loop(0, G, grp, 0)
    pltpu.sync_copy(out_v, out_hbm.at[pl.ds(wid * PW, PW)])


def kernel(triples, entity_embeddings, relation_embeddings):
    tr = triples.astype(jnp.int32)
    heads = tr[:, 0].reshape(NW, NCH, CH)
    tails = tr[:, 1].reshape(NW, NCH, CH)
    rels = tr[:, 2].reshape(NW, NCH, CH)
    ent_f = entity_embeddings.T.reshape(D * E)
    rel_f = relation_embeddings.T.reshape(D * E)
    mesh = plsc.VectorSubcoreMesh(core_axis_name="c", subcore_axis_name="s")
    f = pl.kernel(
        _body,
        out_type=jax.ShapeDtypeStruct((B,), jnp.float32),
        mesh=mesh,
        compiler_params=pltpu.CompilerParams(
            needs_layout_passes=False, use_tc_tiling_on_sc=False),
        scratch_types=[
            pltpu.VMEM((NCH, CH), jnp.int32),
            pltpu.VMEM((NCH, CH), jnp.int32),
            pltpu.VMEM((NCH, CH), jnp.int32),
            pltpu.VMEM((NCH, CH), jnp.int32),
            pltpu.VMEM((NCH, CH), jnp.int32),
            pltpu.VMEM((NCH, CH), jnp.int32),
            pltpu.VMEM((D, PW), jnp.float32),
            pltpu.VMEM((D, PW), jnp.float32),
            pltpu.VMEM((D, PW), jnp.float32),
            pltpu.VMEM((PW,), jnp.float32),
            pltpu.SemaphoreType.DMA,
        ],
    )
    return f(heads, tails, rels, ent_f, rel_f)


# speedup vs baseline: 9.0978x; 9.0978x over previous
import jax
import jax.numpy as jnp
from jax import lax
from jax.experimental import pallas as pl
from jax.experimental.pallas import tpu as pltpu
from jax.experimental.pallas import tpu_sc as plsc

B = 16384
D = 64
DP = 128
L = 16
NC = 2
NS = 16
NW = NC * NS
PW = B // NW
CH = 128
NCH = PW // CH


def _rsqrt(x):
    i = lax.bitcast_convert_type(x, jnp.int32)
    i = jnp.int32(0x5F3759DF) - lax.shift_right_arithmetic(i, 1)
    y = lax.bitcast_convert_type(i, jnp.float32)
    for _ in range(3):
        t = (x * y) * y
        y = y * (jnp.float32(1.5) - jnp.float32(0.5) * t)
    return y


def _body(head_hbm, tail_hbm, rel_hbm, ent_hbm, remb_hbm, out_hbm,
          idx_h, idx_t, idx_r, rows_h, rows_t, rows_r, out_v, sem):
    wid = lax.axis_index("s") * NC + lax.axis_index("c")

    pltpu.sync_copy(head_hbm.at[wid], idx_h)
    pltpu.sync_copy(tail_hbm.at[wid], idx_t)
    pltpu.sync_copy(rel_hbm.at[wid], idx_r)

    iot = lax.iota(jnp.int32, L)

    def chunk(c, carry):
        cp_h = pltpu.async_copy(ent_hbm.at[idx_h.at[c]], rows_h, sem)
        cp_t = pltpu.async_copy(ent_hbm.at[idx_t.at[c]], rows_t, sem)
        cp_r = pltpu.async_copy(remb_hbm.at[idx_r.at[c]], rows_r, sem)
        cp_h.wait()
        cp_t.wait()
        cp_r.wait()

        def grp(j, carry2):
            rowv = j * L + iot

            def facc(f, hh_tt):
                hh, tt = hh_tt
                fv = jnp.full((L,), 0, jnp.int32) + f
                hv = plsc.load_gather(rows_h, [rowv, fv])
                tv = plsc.load_gather(rows_t, [rowv, fv])
                return hh + hv * hv, tt + tv * tv

            z = jnp.zeros((L,), jnp.float32)
            hh, tt = lax.fori_loop(0, D, facc, (z, z))
            ih = jnp.minimum(_rsqrt(hh), jnp.float32(1e12))
            it = jnp.minimum(_rsqrt(tt), jnp.float32(1e12))

            def fsc(f, acc):
                fv = jnp.full((L,), 0, jnp.int32) + f
                hv = plsc.load_gather(rows_h, [rowv, fv])
                tv = plsc.load_gather(rows_t, [rowv, fv])
                rv = plsc.load_gather(rows_r, [rowv, fv])
                return acc + jnp.abs(hv * ih + rv - tv * it)

            out_v[pl.ds(c * CH + j * L, L)] = lax.fori_loop(0, D, fsc, z)
            return carry2

        lax.fori_loop(0, CH // L, grp, 0)
        return carry

    lax.fori_loop(0, NCH, chunk, 0)
    pltpu.sync_copy(out_v, out_hbm.at[pl.ds(wid * PW, PW)])


def kernel(triples, entity_embeddings, relation_embeddings):
    tr = triples.astype(jnp.int32)
    heads = tr[:, 0].reshape(NW, NCH, CH)
    tails = tr[:, 1].reshape(NW, NCH, CH)
    rels = tr[:, 2].reshape(NW, NCH, CH)
    ent_p = jnp.pad(entity_embeddings, ((0, 0), (0, DP - D)))
    rel_p = jnp.pad(relation_embeddings, ((0, 0), (0, DP - D)))
    mesh = plsc.VectorSubcoreMesh(core_axis_name="c", subcore_axis_name="s")
    f = pl.kernel(
        _body,
        out_type=jax.ShapeDtypeStruct((B,), jnp.float32),
        mesh=mesh,
        compiler_params=pltpu.CompilerParams(
            needs_layout_passes=False, use_tc_tiling_on_sc=False),
        scratch_types=[
            pltpu.VMEM((NCH, CH), jnp.int32),
            pltpu.VMEM((NCH, CH), jnp.int32),
            pltpu.VMEM((NCH, CH), jnp.int32),
            pltpu.VMEM((CH, DP), jnp.float32),
            pltpu.VMEM((CH, DP), jnp.float32),
            pltpu.VMEM((CH, DP), jnp.float32),
            pltpu.VMEM((PW,), jnp.float32),
            pltpu.SemaphoreType.DMA,
        ],
    )
    return f(heads, tails, rels, ent_p, rel_p)


# speedup vs baseline: 9.5732x; 1.0523x over previous
import jax
import jax.numpy as jnp
from jax import lax
from jax.experimental import pallas as pl
from jax.experimental.pallas import tpu as pltpu
from jax.experimental.pallas import tpu_sc as plsc

B = 16384
D = 64
DP = 128
L = 16
NC = 2
NS = 16
NW = NC * NS
PW = B // NW
CH = 128
NCH = PW // CH


def _rsqrt(x):
    i = lax.bitcast_convert_type(x, jnp.int32)
    i = jnp.int32(0x5F3759DF) - lax.shift_right_arithmetic(i, 1)
    y = lax.bitcast_convert_type(i, jnp.float32)
    for _ in range(3):
        t = (x * y) * y
        y = y * (jnp.float32(1.5) - jnp.float32(0.5) * t)
    return y


def _lane_sum(v, iot):
    for s in (1, 2, 4, 8):
        v = v + jnp.take_along_axis(v, iot ^ s, axis=0,
                                    mode="promise_in_bounds")
    return v


def _body(head_hbm, tail_hbm, rel_hbm, ent_hbm, remb_hbm, out_hbm,
          idx_h, idx_t, idx_r, rows_h, rows_t, rows_r, out_v, sem):
    wid = lax.axis_index("s") * NC + lax.axis_index("c")

    pltpu.sync_copy(head_hbm.at[wid], idx_h)
    pltpu.sync_copy(tail_hbm.at[wid], idx_t)
    pltpu.sync_copy(rel_hbm.at[wid], idx_r)

    iot = lax.iota(jnp.int32, L)
    lane0 = iot == 0

    def chunk(c, carry):
        cp_h = pltpu.async_copy(ent_hbm.at[idx_h.at[c]], rows_h, sem)
        cp_t = pltpu.async_copy(ent_hbm.at[idx_t.at[c]], rows_t, sem)
        cp_r = pltpu.async_copy(remb_hbm.at[idx_r.at[c]], rows_r, sem)
        cp_h.wait()
        cp_t.wait()
        cp_r.wait()

        def tri(i, carry2):
            h = [rows_h[i, pl.ds(k * L, L)] for k in range(D // L)]
            t = [rows_t[i, pl.ds(k * L, L)] for k in range(D // L)]
            r = [rows_r[i, pl.ds(k * L, L)] for k in range(D // L)]
            hh = h[0] * h[0] + h[1] * h[1] + h[2] * h[2] + h[3] * h[3]
            tt = t[0] * t[0] + t[1] * t[1] + t[2] * t[2] + t[3] * t[3]
            ih = jnp.minimum(_rsqrt(_lane_sum(hh, iot)), jnp.float32(1e12))
            it = jnp.minimum(_rsqrt(_lane_sum(tt, iot)), jnp.float32(1e12))
            acc = jnp.abs(h[0] * ih + r[0] - t[0] * it)
            for k in range(1, D // L):
                acc = acc + jnp.abs(h[k] * ih + r[k] - t[k] * it)
            res = _lane_sum(acc, iot)
            plsc.store_scatter(out_v, [jnp.full((L,), c * CH + i, jnp.int32)],
                               res, mask=lane0)
            return carry2

        lax.fori_loop(0, CH, tri, 0)
        return carry

    lax.fori_loop(0, NCH, chunk, 0)
    pltpu.sync_copy(out_v, out_hbm.at[pl.ds(wid * PW, PW)])


def kernel(triples, entity_embeddings, relation_embeddings):
    tr = triples.astype(jnp.int32)
    heads = tr[:, 0].reshape(NW, NCH, CH)
    tails = tr[:, 1].reshape(NW, NCH, CH)
    rels = tr[:, 2].reshape(NW, NCH, CH)
    ent_p = jnp.pad(entity_embeddings, ((0, 0), (0, DP - D)))
    rel_p = jnp.pad(relation_embeddings, ((0, 0), (0, DP - D)))
    mesh = plsc.VectorSubcoreMesh(core_axis_name="c", subcore_axis_name="s")
    f = pl.kernel(
        _body,
        out_type=jax.ShapeDtypeStruct((B,), jnp.float32),
        mesh=mesh,
        compiler_params=pltpu.CompilerParams(
            needs_layout_passes=False, use_tc_tiling_on_sc=False),
        scratch_types=[
            pltpu.VMEM((NCH, CH), jnp.int32),
            pltpu.VMEM((NCH, CH), jnp.int32),
            pltpu.VMEM((NCH, CH), jnp.int32),
            pltpu.VMEM((CH, DP), jnp.float32),
            pltpu.VMEM((CH, DP), jnp.float32),
            pltpu.VMEM((CH, DP), jnp.float32),
            pltpu.VMEM((PW,), jnp.float32),
            pltpu.SemaphoreType.DMA,
        ],
    )
    return f(heads, tails, rels, ent_p, rel_p)
